# baseline (speedup 1.0000x reference)
import functools

import numpy as _np

import jax
import jax.numpy as jnp
from jax import lax
from jax.experimental import pallas as pl
from jax.experimental.pallas import tpu as pltpu
from jax.experimental.pallas import tpu_sc as plsc

N = 10000
E = 320000
D = 128
H = 128

NC = 2
NS = 16
NW = NC * NS
EPW = E // NW

GC = 80
NCHUNK = EPW // GC

@functools.lru_cache(maxsize=None)
def _mesh():
    return plsc.VectorSubcoreMesh(
        core_axis_name="c", subcore_axis_name="s", num_cores=NC, num_subcores=NS)


def _pre_body(nf_ref, we1a_ref, we1b_ref, wn1a_ref, ps_ref, pr_ref, qn_ref):
    nf = nf_ref[...]
    ps_ref[...] = jnp.dot(nf, we1a_ref[...], preferred_element_type=jnp.float32)
    pr_ref[...] = jnp.dot(nf, we1b_ref[...], preferred_element_type=jnp.float32)
    qn_ref[...] = jnp.dot(nf, wn1a_ref[...], preferred_element_type=jnp.float32)


def _pre(nf, we1a, we1b, wn1a):
    return pl.pallas_call(
        _pre_body,
        out_shape=(jax.ShapeDtypeStruct((N, D), jnp.float32),
                   jax.ShapeDtypeStruct((N, D), jnp.float32),
                   jax.ShapeDtypeStruct((N, D), jnp.float32)),
    )(nf, we1a, we1b, wn1a)


def _gather_body(ps_hbm, pr_hbm, s_hbm, r_hbm, out_hbm,
                 sidx_all, ridx_all, a0, a1, a2, a3, b0, b1, b2, b3, c0, c1,
                 gsem0, gsem1, gsem2, gsem3, ssem0, ssem1):
    sid = lax.axis_index("s")
    wid = lax.axis_index("c") * NS + sid
    base0 = wid * EPW
    abuf = (a0, a1, a2, a3)
    bbuf = (b0, b1, b2, b3)
    cbuf = (c0, c1)
    gsem = (gsem0, gsem1, gsem2, gsem3)
    ssem = (ssem0, ssem1)

    pltpu.sync_copy(s_hbm.at[wid], sidx_all)
    pltpu.sync_copy(r_hbm.at[wid], ridx_all)

    def fire_g(k, b):
        sl = pl.ds(k * GC, GC)
        pltpu.async_copy(ps_hbm.at[sidx_all.at[sl]], abuf[b], gsem[b])
        pltpu.async_copy(pr_hbm.at[ridx_all.at[sl]], bbuf[b], gsem[b])

    def wait_g(b):
        pltpu.make_async_copy(ps_hbm.at[pl.ds(0, GC)], abuf[b], gsem[b]).wait()
        pltpu.make_async_copy(pr_hbm.at[pl.ds(0, GC)], bbuf[b], gsem[b]).wait()

    half_ulp = jnp.int32(32768)
    hi_mask = jnp.int32(-65536)

    def vadd(b, cb):
        def row(i, carry):
            for j in range(D // 32):
                lo = abuf[b][i, pl.ds(32 * j, 16)] + bbuf[b][i, pl.ds(32 * j, 16)]
                hi = abuf[b][i, pl.ds(32 * j + 16, 16)] + bbuf[b][i, pl.ds(32 * j + 16, 16)]
                li = lax.bitcast_convert_type(lo, jnp.int32) + half_ulp
                hj = lax.bitcast_convert_type(hi, jnp.int32) + half_ulp
                cbuf[cb][i, pl.ds(16 * j, 16)] = (
                    lax.shift_right_logical(li, 16) | (hj & hi_mask))
            return carry
        lax.fori_loop(0, GC, row, 0, unroll=4)

    def fire_s(k, cb):
        pltpu.async_copy(cbuf[cb], out_hbm.at[pl.ds(base0 + k * GC, GC)], ssem[cb])

    def wait_s(cb):
        pltpu.make_async_copy(cbuf[cb], out_hbm.at[pl.ds(base0, GC)], ssem[cb]).wait()

    RD = 4
    for b in range(RD):
        fire_g(b, b)

    def quad(i, carry):
        for b in range(RD):
            k = RD * i + b
            cb = b % 2
            wait_g(b)

            @pl.when(k >= 2)
            def _():
                wait_s(cb)

            vadd(b, cb)
            fire_s(k, cb)

            @pl.when(k + RD <= NCHUNK - 1)
            def _():
                fire_g(k + RD, b)
        return carry

    lax.fori_loop(0, NCHUNK // RD, quad, 0)

    k_tail = (NCHUNK // RD) * RD
    wait_g(0)
    wait_s(0)
    vadd(0, 0)
    fire_s(k_tail, 0)
    wait_s(1)
    wait_s(0)


@functools.lru_cache(maxsize=None)
def _gather_kernel():
    gbuf = pltpu.VMEM((GC, D), jnp.float32)
    cbuf = pltpu.VMEM((GC, D // 2), jnp.int32)
    return pl.kernel(
        _gather_body,
        out_type=jax.ShapeDtypeStruct((E, D // 2), jnp.int32),
        mesh=_mesh(),
        scratch_types=(
            [pltpu.VMEM((EPW,), jnp.int32)] * 2
            + [gbuf] * 8
            + [cbuf] * 2
            + [pltpu.SemaphoreType.DMA] * 6
        ),
    )


def _edge_body(g_ref, ef_ref, we1c_ref, be1_ref, we2_ref, be2_ref,
               ge_ref, bbe_ref, oe_ref, ne_ref):
    ef = ef_ref[...]
    x = g_ref[...].astype(jnp.float32) + jnp.dot(
        ef, we1c_ref[...], preferred_element_type=jnp.float32) + be1_ref[...]
    h = jnp.maximum(x, 0.0)
    h2 = jnp.dot(h, we2_ref[...], preferred_element_type=jnp.float32) + be2_ref[...]
    mu = jnp.mean(h2, axis=-1, keepdims=True)
    var = jnp.mean((h2 - mu) * (h2 - mu), axis=-1, keepdims=True)
    ne = (h2 - mu) * lax.rsqrt(var + 1e-5) * ge_ref[...] + bbe_ref[...]
    ne_ref[...] = ne
    oe_ref[...] = ef + ne


def _edge_mlp(g, ef, we1c, be1, we2, be2, ge, bbe):
    BE = 2000
    grid = E // BE
    row_spec = pl.BlockSpec((BE, D), lambda i: (i, 0))
    w_spec = pl.BlockSpec((D, H), lambda i: (0, 0))
    v_spec = pl.BlockSpec((1, D), lambda i: (0, 0))
    out = jax.ShapeDtypeStruct((E, D), jnp.float32)
    return pl.pallas_call(
        _edge_body,
        grid=(grid,),
        in_specs=[row_spec, row_spec, w_spec, v_spec, w_spec, v_spec, v_spec, v_spec],
        out_specs=(row_spec, row_spec),
        out_shape=(out, out),
    )(g, ef, we1c, be1, we2, be2, ge, bbe)


def _segsum_body(ne_hbm, r_hbm, zeros_hbm, out_hbm,
                 ridx, ld0, ld1, agg, lsem0, lsem1, ssem0, ssem1):
    cid = lax.axis_index("c")
    sid = lax.axis_index("s")
    ld = (ld0, ld1)
    lsem = (lsem0, lsem1)
    ssem = (ssem0, ssem1)

    base0 = (cid * NS + sid) * EPW
    wid = cid * NS + sid

    pltpu.sync_copy(r_hbm.at[wid], ridx)

    @pl.when(sid == 0)
    def _():
        pltpu.sync_copy(zeros_hbm, agg)

    plsc.subcore_barrier()

    def fire_l(k, b):
        pltpu.async_copy(ne_hbm.at[pl.ds(base0 + k * GC, GC)], ld[b], lsem[b])

    def wait_l(b):
        pltpu.make_async_copy(ne_hbm.at[pl.ds(base0, GC)], ld[b], lsem[b]).wait()

    def fire_sc(k, b):
        pltpu.async_copy(ld[b], agg.at[ridx.at[k]], ssem[b], add=True)

    def wait_sc(b):
        pltpu.make_async_copy(ld[b], agg.at[ridx.at[0]], ssem[b]).wait()

    fire_l(0, 0)

    def pair(i, carry):
        for b in (0, 1):
            k = 2 * i + b

            @pl.when(k >= 1)
            def _():
                wait_sc(1 - b)

            @pl.when(k + 1 <= NCHUNK - 1)
            def _():
                fire_l(k + 1, 1 - b)

            wait_l(b)
            fire_sc(k, b)
        return carry

    lax.fori_loop(0, NCHUNK // 2, pair, 0)

    wait_sc(1)
    wait_l(0)
    fire_sc(NCHUNK - 1, 0)
    wait_sc(0)

    plsc.subcore_barrier()

    rows = 624
    sl = pl.ds(sid * rows, rows)
    pltpu.sync_copy(agg.at[sl], out_hbm.at[cid].at[sl])

    @pl.when(sid == 0)
    def _():
        tail = pl.ds(NS * rows, N - NS * rows)
        pltpu.sync_copy(agg.at[tail], out_hbm.at[cid].at[tail])


@functools.lru_cache(maxsize=None)
def _segsum_kernel():
    return pl.kernel(
        _segsum_body,
        out_type=jax.ShapeDtypeStruct((NC, N, D), jnp.float32),
        mesh=_mesh(),
        scratch_types=[
            pltpu.VMEM((NCHUNK, GC), jnp.int32),
            pltpu.VMEM((GC, D), jnp.float32),
            pltpu.VMEM((GC, D), jnp.float32),
            pltpu.VMEM_SHARED((N, D), jnp.float32),
            pltpu.SemaphoreType.DMA,
            pltpu.SemaphoreType.DMA,
            pltpu.SemaphoreType.DMA,
            pltpu.SemaphoreType.DMA,
        ],
    )


def _node_body(nf_ref, qn_ref, p0_ref, p1_ref, wn1b_ref, bn1_ref,
               wn2_ref, bn2_ref, gn_ref, bbn_ref, on_ref):
    agg = p0_ref[...] + p1_ref[...]
    x = qn_ref[...] + jnp.dot(agg, wn1b_ref[...],
                              preferred_element_type=jnp.float32) + bn1_ref[...]
    h = jnp.maximum(x, 0.0)
    h2 = jnp.dot(h, wn2_ref[...], preferred_element_type=jnp.float32) + bn2_ref[...]
    mu = jnp.mean(h2, axis=-1, keepdims=True)
    var = jnp.mean((h2 - mu) * (h2 - mu), axis=-1, keepdims=True)
    nn = (h2 - mu) * lax.rsqrt(var + 1e-5) * gn_ref[...] + bbn_ref[...]
    on_ref[...] = nf_ref[...] + nn


def _node_mlp(nf, qn, p0, p1, wn1b, bn1, wn2, bn2, gn, bbn):
    BN = 2000
    grid = N // BN
    row_spec = pl.BlockSpec((BN, D), lambda i: (i, 0))
    w_spec = pl.BlockSpec((D, H), lambda i: (0, 0))
    v_spec = pl.BlockSpec((1, D), lambda i: (0, 0))
    return pl.pallas_call(
        _node_body,
        grid=(grid,),
        in_specs=[row_spec, row_spec, row_spec, row_spec,
                  w_spec, v_spec, w_spec, v_spec, v_spec, v_spec],
        out_specs=row_spec,
        out_shape=jax.ShapeDtypeStruct((N, D), jnp.float32),
    )(nf, qn, p0, p1, wn1b, bn1, wn2, bn2, gn, bbn)


def kernel(node_features, senders, receivers, edge_features,
           We1, be1, We2, be2, ge, bbe, Wn1, bn1, Wn2, bn2, gn, bbn):
    senders = senders.astype(jnp.int32)
    receivers = receivers.astype(jnp.int32)

    we1a, we1b, we1c = We1[0:D], We1[D:2 * D], We1[2 * D:3 * D]
    wn1a, wn1b = Wn1[0:D], Wn1[D:2 * D]

    s2 = senders.reshape(NW, EPW)
    r2 = receivers.reshape(NW, EPW)
    r3 = receivers.reshape(NW, NCHUNK, GC)

    ps_tbl, pr_tbl, qn = _pre(node_features, we1a, we1b, wn1a)
    g_i32 = _gather_kernel()(ps_tbl, pr_tbl, s2, r2)
    g_bf = lax.bitcast_convert_type(g_i32, jnp.bfloat16).reshape(E, D)

    perm = jnp.asarray(
        _np.arange(D).reshape(D // 32, 2, 16).transpose(0, 2, 1).reshape(D))
    out_edges, new_edge = _edge_mlp(
        g_bf, edge_features, we1c[:, perm],
        be1[perm].reshape(1, D), We2[perm, :], be2.reshape(1, D),
        ge.reshape(1, D), bbe.reshape(1, D))
    zeros = jnp.zeros((N, D), jnp.float32)
    parts = _segsum_kernel()(new_edge, r3, zeros)
    out_nodes = _node_mlp(
        node_features, qn, parts[0], parts[1],
        wn1b, bn1.reshape(1, D), Wn2, bn2.reshape(1, D),
        gn.reshape(1, D), bbn.reshape(1, D))
    return (out_nodes, out_edges)


# speedup vs baseline: 1.8211x; 1.8211x over previous
import functools

import numpy as _np

import jax
import jax.numpy as jnp
from jax import lax
from jax.experimental import pallas as pl
from jax.experimental.pallas import tpu as pltpu
from jax.experimental.pallas import tpu_sc as plsc

N = 10000
E = 320000
D = 128
H = 128

NC = 2
NS = 16
NW = NC * NS
EPW = E // NW

GC = 80
NCHUNK = EPW // GC

@functools.lru_cache(maxsize=None)
def _mesh():
    return plsc.VectorSubcoreMesh(
        core_axis_name="c", subcore_axis_name="s", num_cores=NC, num_subcores=NS)


def _pre_body(nf_ref, we1a_ref, we1b_ref, wn1a_ref, ps_ref, pr_ref, qn_ref):
    nf = nf_ref[...]
    ps_ref[...] = jnp.dot(nf, we1a_ref[...], preferred_element_type=jnp.float32)
    pr_ref[...] = jnp.dot(nf, we1b_ref[...], preferred_element_type=jnp.float32)
    qn_ref[...] = jnp.dot(nf, wn1a_ref[...], preferred_element_type=jnp.float32)


def _pre(nf, we1a, we1b, wn1a):
    return pl.pallas_call(
        _pre_body,
        out_shape=(jax.ShapeDtypeStruct((N, D), jnp.float32),
                   jax.ShapeDtypeStruct((N, D), jnp.float32),
                   jax.ShapeDtypeStruct((N, D), jnp.float32)),
    )(nf, we1a, we1b, wn1a)


def _gather_body(ps_hbm, pr_hbm, s_hbm, r_hbm, out_hbm,
                 sidx_all, ridx_all, a0, a1, a2, a3, b0, b1, b2, b3, c0, c1,
                 gsem0, gsem1, gsem2, gsem3, ssem0, ssem1):
    sid = lax.axis_index("s")
    wid = lax.axis_index("c") * NS + sid
    base0 = wid * EPW
    abuf = (a0, a1, a2, a3)
    bbuf = (b0, b1, b2, b3)
    cbuf = (c0, c1)
    gsem = (gsem0, gsem1, gsem2, gsem3)
    ssem = (ssem0, ssem1)

    pltpu.sync_copy(s_hbm.at[wid], sidx_all)
    pltpu.sync_copy(r_hbm.at[wid], ridx_all)

    def fire_g(k, b):
        sl = pl.ds(k * GC, GC)
        pltpu.async_copy(ps_hbm.at[sidx_all.at[sl]], abuf[b], gsem[b])
        pltpu.async_copy(pr_hbm.at[ridx_all.at[sl]], bbuf[b], gsem[b])

    def wait_g(b):
        pltpu.make_async_copy(ps_hbm.at[pl.ds(0, GC)], abuf[b], gsem[b]).wait()
        pltpu.make_async_copy(pr_hbm.at[pl.ds(0, GC)], bbuf[b], gsem[b]).wait()

    half_ulp = jnp.int32(32768)
    hi_mask = jnp.int32(-65536)

    def vadd(b, cb):
        def row(i, carry):
            for j in range(D // 32):
                lo = abuf[b][i, pl.ds(32 * j, 16)] + bbuf[b][i, pl.ds(32 * j, 16)]
                hi = abuf[b][i, pl.ds(32 * j + 16, 16)] + bbuf[b][i, pl.ds(32 * j + 16, 16)]
                li = lax.bitcast_convert_type(lo, jnp.int32) + half_ulp
                hj = lax.bitcast_convert_type(hi, jnp.int32) + half_ulp
                cbuf[cb][i, pl.ds(16 * j, 16)] = (
                    lax.shift_right_logical(li, 16) | (hj & hi_mask))
            return carry
        lax.fori_loop(0, GC, row, 0, unroll=4)

    def fire_s(k, cb):
        pltpu.async_copy(cbuf[cb], out_hbm.at[pl.ds(base0 + k * GC, GC)], ssem[cb])

    def wait_s(cb):
        pltpu.make_async_copy(cbuf[cb], out_hbm.at[pl.ds(base0, GC)], ssem[cb]).wait()

    RD = 4
    for b in range(RD):
        fire_g(b, b)

    def quad(i, carry):
        for b in range(RD):
            k = RD * i + b
            cb = b % 2
            wait_g(b)

            @pl.when(k >= 2)
            def _():
                wait_s(cb)

            vadd(b, cb)
            fire_s(k, cb)

            @pl.when(k + RD <= NCHUNK - 1)
            def _():
                fire_g(k + RD, b)
        return carry

    lax.fori_loop(0, NCHUNK // RD, quad, 0)

    k_tail = (NCHUNK // RD) * RD
    wait_g(0)
    wait_s(0)
    vadd(0, 0)
    fire_s(k_tail, 0)
    wait_s(1)
    wait_s(0)


@functools.lru_cache(maxsize=None)
def _gather_kernel():
    gbuf = pltpu.VMEM((GC, D), jnp.float32)
    cbuf = pltpu.VMEM((GC, D // 2), jnp.int32)
    return pl.kernel(
        _gather_body,
        out_type=jax.ShapeDtypeStruct((E, D // 2), jnp.int32),
        mesh=_mesh(),
        scratch_types=(
            [pltpu.VMEM((EPW,), jnp.int32)] * 2
            + [gbuf] * 8
            + [cbuf] * 2
            + [pltpu.SemaphoreType.DMA] * 6
        ),
    )


def _edge_body(g_ref, ef_ref, w1lo_ref, w1hi_ref, blo_ref, bhi_ref,
               w2lo_ref, w2hi_ref, be2_ref, ge_ref, bbe_ref, oe_ref, ne_ref):
    ef = ef_ref[...]
    g32 = g_ref[...]
    hi_mask = jnp.int32(-65536)
    xl = (lax.bitcast_convert_type(g32 << 16, jnp.float32)
          + jnp.dot(ef, w1lo_ref[...], preferred_element_type=jnp.float32)
          + blo_ref[...])
    xh = (lax.bitcast_convert_type(g32 & hi_mask, jnp.float32)
          + jnp.dot(ef, w1hi_ref[...], preferred_element_type=jnp.float32)
          + bhi_ref[...])
    h2 = (jnp.dot(jnp.maximum(xl, 0.0), w2lo_ref[...],
                  preferred_element_type=jnp.float32)
          + jnp.dot(jnp.maximum(xh, 0.0), w2hi_ref[...],
                    preferred_element_type=jnp.float32)
          + be2_ref[...])
    mu = jnp.mean(h2, axis=-1, keepdims=True)
    var = jnp.mean((h2 - mu) * (h2 - mu), axis=-1, keepdims=True)
    ne = (h2 - mu) * lax.rsqrt(var + 1e-5) * ge_ref[...] + bbe_ref[...]
    ne_ref[...] = ne
    oe_ref[...] = ef + ne


def _edge_mlp(g, ef, w1lo, w1hi, blo, bhi, w2lo, w2hi, be2, ge, bbe):
    BE = 2000
    grid = E // BE
    row_spec = pl.BlockSpec((BE, D), lambda i: (i, 0))
    g_spec = pl.BlockSpec((BE, D // 2), lambda i: (i, 0))
    wl_spec = pl.BlockSpec((D, H // 2), lambda i: (0, 0))
    w2_spec = pl.BlockSpec((H // 2, D), lambda i: (0, 0))
    hv_spec = pl.BlockSpec((1, H // 2), lambda i: (0, 0))
    v_spec = pl.BlockSpec((1, D), lambda i: (0, 0))
    out = jax.ShapeDtypeStruct((E, D), jnp.float32)
    return pl.pallas_call(
        _edge_body,
        grid=(grid,),
        in_specs=[g_spec, row_spec, wl_spec, wl_spec, hv_spec, hv_spec,
                  w2_spec, w2_spec, v_spec, v_spec, v_spec],
        out_specs=(row_spec, row_spec),
        out_shape=(out, out),
    )(g, ef, w1lo, w1hi, blo, bhi, w2lo, w2hi, be2, ge, bbe)


def _segsum_body(ne_hbm, r_hbm, zeros_hbm, out_hbm,
                 ridx, ld0, ld1, agg, lsem0, lsem1, ssem0, ssem1):
    cid = lax.axis_index("c")
    sid = lax.axis_index("s")
    ld = (ld0, ld1)
    lsem = (lsem0, lsem1)
    ssem = (ssem0, ssem1)

    base0 = (cid * NS + sid) * EPW
    wid = cid * NS + sid

    pltpu.sync_copy(r_hbm.at[wid], ridx)

    @pl.when(sid == 0)
    def _():
        pltpu.sync_copy(zeros_hbm, agg)

    plsc.subcore_barrier()

    def fire_l(k, b):
        pltpu.async_copy(ne_hbm.at[pl.ds(base0 + k * GC, GC)], ld[b], lsem[b])

    def wait_l(b):
        pltpu.make_async_copy(ne_hbm.at[pl.ds(base0, GC)], ld[b], lsem[b]).wait()

    def fire_sc(k, b):
        pltpu.async_copy(ld[b], agg.at[ridx.at[k]], ssem[b], add=True)

    def wait_sc(b):
        pltpu.make_async_copy(ld[b], agg.at[ridx.at[0]], ssem[b]).wait()

    fire_l(0, 0)

    def pair(i, carry):
        for b in (0, 1):
            k = 2 * i + b

            @pl.when(k >= 1)
            def _():
                wait_sc(1 - b)

            @pl.when(k + 1 <= NCHUNK - 1)
            def _():
                fire_l(k + 1, 1 - b)

            wait_l(b)
            fire_sc(k, b)
        return carry

    lax.fori_loop(0, NCHUNK // 2, pair, 0)

    wait_sc(1)
    wait_l(0)
    fire_sc(NCHUNK - 1, 0)
    wait_sc(0)

    plsc.subcore_barrier()

    rows = 624
    sl = pl.ds(sid * rows, rows)
    pltpu.sync_copy(agg.at[sl], out_hbm.at[cid].at[sl])

    @pl.when(sid == 0)
    def _():
        tail = pl.ds(NS * rows, N - NS * rows)
        pltpu.sync_copy(agg.at[tail], out_hbm.at[cid].at[tail])


@functools.lru_cache(maxsize=None)
def _segsum_kernel():
    return pl.kernel(
        _segsum_body,
        out_type=jax.ShapeDtypeStruct((NC, N, D), jnp.float32),
        mesh=_mesh(),
        scratch_types=[
            pltpu.VMEM((NCHUNK, GC), jnp.int32),
            pltpu.VMEM((GC, D), jnp.float32),
            pltpu.VMEM((GC, D), jnp.float32),
            pltpu.VMEM_SHARED((N, D), jnp.float32),
            pltpu.SemaphoreType.DMA,
            pltpu.SemaphoreType.DMA,
            pltpu.SemaphoreType.DMA,
            pltpu.SemaphoreType.DMA,
        ],
    )


def _node_body(nf_ref, qn_ref, p0_ref, p1_ref, wn1b_ref, bn1_ref,
               wn2_ref, bn2_ref, gn_ref, bbn_ref, on_ref):
    agg = p0_ref[...] + p1_ref[...]
    x = qn_ref[...] + jnp.dot(agg, wn1b_ref[...],
                              preferred_element_type=jnp.float32) + bn1_ref[...]
    h = jnp.maximum(x, 0.0)
    h2 = jnp.dot(h, wn2_ref[...], preferred_element_type=jnp.float32) + bn2_ref[...]
    mu = jnp.mean(h2, axis=-1, keepdims=True)
    var = jnp.mean((h2 - mu) * (h2 - mu), axis=-1, keepdims=True)
    nn = (h2 - mu) * lax.rsqrt(var + 1e-5) * gn_ref[...] + bbn_ref[...]
    on_ref[...] = nf_ref[...] + nn


def _node_mlp(nf, qn, p0, p1, wn1b, bn1, wn2, bn2, gn, bbn):
    BN = 2000
    grid = N // BN
    row_spec = pl.BlockSpec((BN, D), lambda i: (i, 0))
    w_spec = pl.BlockSpec((D, H), lambda i: (0, 0))
    v_spec = pl.BlockSpec((1, D), lambda i: (0, 0))
    return pl.pallas_call(
        _node_body,
        grid=(grid,),
        in_specs=[row_spec, row_spec, row_spec, row_spec,
                  w_spec, v_spec, w_spec, v_spec, v_spec, v_spec],
        out_specs=row_spec,
        out_shape=jax.ShapeDtypeStruct((N, D), jnp.float32),
    )(nf, qn, p0, p1, wn1b, bn1, wn2, bn2, gn, bbn)


def kernel(node_features, senders, receivers, edge_features,
           We1, be1, We2, be2, ge, bbe, Wn1, bn1, Wn2, bn2, gn, bbn):
    senders = senders.astype(jnp.int32)
    receivers = receivers.astype(jnp.int32)

    we1a, we1b, we1c = We1[0:D], We1[D:2 * D], We1[2 * D:3 * D]
    wn1a, wn1b = Wn1[0:D], Wn1[D:2 * D]

    s2 = senders.reshape(NW, EPW)
    r2 = receivers.reshape(NW, EPW)
    r3 = receivers.reshape(NW, NCHUNK, GC)

    ps_tbl, pr_tbl, qn = _pre(node_features, we1a, we1b, wn1a)
    g_i32 = _gather_kernel()(ps_tbl, pr_tbl, s2, r2)

    lanes = _np.arange(D // 2)
    perm_lo = jnp.asarray((lanes // 16) * 32 + lanes % 16)
    perm_hi = perm_lo + 16
    out_edges, new_edge = _edge_mlp(
        g_i32, edge_features, we1c[:, perm_lo], we1c[:, perm_hi],
        be1[perm_lo].reshape(1, D // 2), be1[perm_hi].reshape(1, D // 2),
        We2[perm_lo, :], We2[perm_hi, :], be2.reshape(1, D),
        ge.reshape(1, D), bbe.reshape(1, D))
    zeros = jnp.zeros((N, D), jnp.float32)
    parts = _segsum_kernel()(new_edge, r3, zeros)
    out_nodes = _node_mlp(
        node_features, qn, parts[0], parts[1],
        wn1b, bn1.reshape(1, D), Wn2, bn2.reshape(1, D),
        gn.reshape(1, D), bbn.reshape(1, D))
    return (out_nodes, out_edges)
